# baseline (speedup 1.0000x reference)
import functools

import jax
import jax.numpy as jnp
from jax import lax
from jax.experimental import pallas as pl
from jax.experimental.pallas import tpu as pltpu
from jax.experimental.pallas import tpu_sc as plsc

_GRID = (32, 32)
_ALPHA = 1.0
_N_NODES = _GRID[0] * _GRID[1]
_BLK = 512


def _dist_block(z_ref, tw_ref, nodes_ref, q_ref, idx_ref,
                nodes_t_ref, nn_ref):
    @pl.when(pl.program_id(0) == 0)
    def _prologue():
        nt = jnp.transpose(nodes_ref[...], (1, 0))
        nodes_t_ref[...] = nt
        nn_ref[...] = jnp.sum(nt * nt, axis=0, keepdims=True)

    z = z_ref[...]
    tw = tw_ref[...]
    wz = z * tw

    mm = jnp.dot(wz, nodes_t_ref[...],
                 preferred_element_type=jnp.float32)
    rowsq = jnp.sum(wz * wz, axis=1, keepdims=True)
    sq = rowsq - 2.0 * mm + nn_ref[...]
    dists = jnp.sqrt(jnp.maximum(sq, 1e-12))

    q_raw = 1.0 / (1.0 + dists / _ALPHA)
    q_ref[...] = q_raw / jnp.sum(q_raw, axis=1, keepdims=True)

    idx = jnp.argmin(dists, axis=1).astype(jnp.int32)
    idx_ref[...] = idx[:, None]


def _blend_block(z_ref, g_ref, som_ref):
    z = z_ref[...]
    som_ref[...] = z + 0.1 * (g_ref[...] - z)


def _make_sc_gather(n_rows, d):
    info = plsc.get_sparse_core_info()
    nw = info.num_cores * info.num_subcores
    b_per_w = n_rows // nw
    mesh = plsc.VectorSubcoreMesh(core_axis_name="c", subcore_axis_name="s")

    @functools.partial(
        pl.kernel, mesh=mesh,
        out_type=jax.ShapeDtypeStruct((n_rows, d), jnp.float32),
        scratch_types=[
            pltpu.VMEM((b_per_w,), jnp.int32),
            pltpu.VMEM((b_per_w, d), jnp.float32),
            pltpu.SemaphoreType.DMA,
        ],
    )
    def _gather(table_hbm, idx_hbm, out_hbm, idx_v, rows_v, sem):
        wid = lax.axis_index("s") * info.num_cores + lax.axis_index("c")
        base = wid * b_per_w
        pltpu.sync_copy(idx_hbm.at[pl.ds(base, b_per_w)], idx_v)
        pltpu.async_copy(table_hbm.at[idx_v], rows_v, sem).wait()
        pltpu.sync_copy(rows_v, out_hbm.at[pl.ds(base, b_per_w)])

    return _gather


@jax.jit
def kernel(z, nodes, time_weights):
    b, t, d = z.shape
    n_rows = b * t
    z_flat = z.reshape(n_rows, d)
    nodes_flat = nodes.reshape(-1, d)
    tw_col = time_weights[0, -t:, :]

    n_blocks = n_rows // _BLK
    tw_blocks = t // _BLK if t >= _BLK else 1

    q, idx = pl.pallas_call(
        _dist_block,
        grid=(n_blocks,),
        in_specs=[
            pl.BlockSpec((_BLK, d), lambda i: (i, 0)),
            pl.BlockSpec((_BLK, 1), lambda i: (i % tw_blocks, 0)),
            pl.BlockSpec((_N_NODES, d), lambda i: (0, 0)),
        ],
        out_specs=[
            pl.BlockSpec((_BLK, _N_NODES), lambda i: (i, 0)),
            pl.BlockSpec((_BLK, 1), lambda i: (i, 0)),
        ],
        out_shape=[
            jax.ShapeDtypeStruct((n_rows, _N_NODES), jnp.float32),
            jax.ShapeDtypeStruct((n_rows, 1), jnp.int32),
        ],
        scratch_shapes=[
            pltpu.VMEM((d, _N_NODES), jnp.float32),
            pltpu.VMEM((1, _N_NODES), jnp.float32),
        ],
    )(z_flat, tw_col, nodes_flat)

    idx_flat = idx.reshape(n_rows)
    gathered = _make_sc_gather(n_rows, d)(nodes_flat, idx_flat)

    som = pl.pallas_call(
        _blend_block,
        grid=(4,),
        in_specs=[
            pl.BlockSpec((n_rows // 4, d), lambda i: (i, 0)),
            pl.BlockSpec((n_rows // 4, d), lambda i: (i, 0)),
        ],
        out_specs=pl.BlockSpec((n_rows // 4, d), lambda i: (i, 0)),
        out_shape=jax.ShapeDtypeStruct((n_rows, d), jnp.float32),
    )(z_flat, gathered)

    som_z = som.reshape(b, t, d)
    bmu_indices = idx_flat.reshape(b, t)
    return som_z, q, bmu_indices


# speedup vs baseline: 7.0122x; 7.0122x over previous
import functools

import jax
import jax.numpy as jnp
from jax.experimental import pallas as pl
from jax.experimental.pallas import tpu as pltpu

_GRID = (32, 32)
_ALPHA = 1.0
_N_NODES = _GRID[0] * _GRID[1]
_BLK = 512

_DN_T = (((1,), (1,)), ((), ()))


def _som_block(tw_base, tw_blocks, z_ref, tw_ref, nodes_ref,
               som_ref, q_ref, idx_ref, nodes_t_ref, nn_ref):
    @pl.when(pl.program_id(0) == 0)
    def _prologue():
        nt = jnp.transpose(nodes_ref[...], (1, 0))
        nodes_t_ref[...] = nt
        nn_ref[...] = jnp.sum(nt * nt, axis=0, keepdims=True)

    z = z_ref[...]
    start = tw_base + (pl.program_id(0) % tw_blocks) * _BLK
    tw = tw_ref[pl.ds(start, _BLK), :]
    nodes_t = nodes_t_ref[...]
    wz = z * tw

    mm = jnp.dot(wz, nodes_t, preferred_element_type=jnp.float32)
    rowsq = jnp.sum(wz * wz, axis=1, keepdims=True)
    sq = rowsq - 2.0 * mm + nn_ref[...]
    dists = jnp.sqrt(jnp.maximum(sq, 1e-12))

    q_raw = 1.0 / (1.0 + dists / _ALPHA)
    q_ref[...] = q_raw / jnp.sum(q_raw, axis=1, keepdims=True)

    idx = jnp.argmin(dists, axis=1).astype(jnp.int32)
    idx_col = idx[:, None]
    idx_ref[...] = idx_col

    lane = jax.lax.broadcasted_iota(jnp.int32, dists.shape, 1)
    onehot = (lane == idx_col).astype(jnp.float32)
    gathered = jax.lax.dot_general(onehot, nodes_t, _DN_T,
                                   preferred_element_type=jnp.float32)
    som_ref[...] = 0.9 * z + 0.1 * gathered


@jax.jit
def kernel(z, nodes, time_weights):
    b, t, d = z.shape
    n_rows = b * t
    z_flat = z.reshape(n_rows, d)
    nodes_flat = nodes.reshape(-1, d)
    max_seq = time_weights.shape[1]
    tw_full = time_weights.reshape(max_seq, 1)

    n_blocks = n_rows // _BLK
    tw_blocks = t // _BLK if t >= _BLK else 1
    tw_base = max_seq - t

    body = functools.partial(_som_block, tw_base, tw_blocks)

    som, q, idx = pl.pallas_call(
        body,
        grid=(n_blocks,),
        in_specs=[
            pl.BlockSpec((_BLK, d), lambda i: (i, 0)),
            pl.BlockSpec((max_seq, 1), lambda i: (0, 0)),
            pl.BlockSpec((_N_NODES, d), lambda i: (0, 0)),
        ],
        out_specs=[
            pl.BlockSpec((_BLK, d), lambda i: (i, 0)),
            pl.BlockSpec((_BLK, _N_NODES), lambda i: (i, 0)),
            pl.BlockSpec((_BLK, 1), lambda i: (i, 0)),
        ],
        out_shape=[
            jax.ShapeDtypeStruct((n_rows, d), jnp.float32),
            jax.ShapeDtypeStruct((n_rows, _N_NODES), jnp.float32),
            jax.ShapeDtypeStruct((n_rows, 1), jnp.int32),
        ],
        scratch_shapes=[
            pltpu.VMEM((d, _N_NODES), jnp.float32),
            pltpu.VMEM((1, _N_NODES), jnp.float32),
        ],
    )(z_flat, tw_full, nodes_flat)

    som_z = som.reshape(b, t, d)
    bmu_indices = idx[:, 0].reshape(b, t)
    return som_z, q, bmu_indices


# speedup vs baseline: 7.4141x; 1.0573x over previous
import jax
import jax.numpy as jnp
from jax.experimental import pallas as pl
from jax.experimental.pallas import tpu as pltpu

_GRID = (32, 32)
_ALPHA = 1.0
_N_NODES = _GRID[0] * _GRID[1]
_BLK = 512

_DN_T = (((1,), (1,)), ((), ()))


def _som_block(z_ref, tw_ref, nodes_ref, som_ref, q_ref, idx_ref,
               nodes_t_ref, nn_ref):
    @pl.when(pl.program_id(0) == 0)
    def _prologue():
        nt = jnp.transpose(nodes_ref[...], (1, 0))
        nodes_t_ref[...] = nt
        nn_ref[...] = jnp.sum(nt * nt, axis=0, keepdims=True)

    z = z_ref[...]
    tw = tw_ref[...]
    nodes_t = nodes_t_ref[...]
    wz = z * tw

    mm = jnp.dot(wz, nodes_t, preferred_element_type=jnp.float32)
    rowsq = jnp.sum(wz * wz, axis=1, keepdims=True)
    sq = rowsq - 2.0 * mm + nn_ref[...]
    dists = jnp.sqrt(jnp.maximum(sq, 1e-12))

    q_raw = 1.0 / (1.0 + dists / _ALPHA)
    q_ref[...] = q_raw / jnp.sum(q_raw, axis=1, keepdims=True)

    idx = jnp.argmin(dists, axis=1).astype(jnp.int32)
    idx_col = idx[:, None]
    idx_ref[...] = idx_col

    lane = jax.lax.broadcasted_iota(jnp.int32, dists.shape, 1)
    onehot = (lane == idx_col).astype(jnp.float32)
    gathered = jax.lax.dot_general(onehot, nodes_t, _DN_T,
                                   preferred_element_type=jnp.float32)
    som_ref[...] = 0.9 * z + 0.1 * gathered


@jax.jit
def kernel(z, nodes, time_weights):
    b, t, d = z.shape
    n_rows = b * t
    z_flat = z.reshape(n_rows, d)
    nodes_flat = nodes.reshape(-1, d)
    tw_col = time_weights[0, -t:, :]

    n_blocks = n_rows // _BLK
    tw_blocks = t // _BLK if t >= _BLK else 1

    som, q, idx = pl.pallas_call(
        _som_block,
        grid=(n_blocks,),
        in_specs=[
            pl.BlockSpec((_BLK, d), lambda i: (i, 0)),
            pl.BlockSpec((_BLK, 1), lambda i: (i % tw_blocks, 0)),
            pl.BlockSpec((_N_NODES, d), lambda i: (0, 0)),
        ],
        out_specs=[
            pl.BlockSpec((_BLK, d), lambda i: (i, 0)),
            pl.BlockSpec((_BLK, _N_NODES), lambda i: (i, 0)),
            pl.BlockSpec((_BLK, 1), lambda i: (i, 0)),
        ],
        out_shape=[
            jax.ShapeDtypeStruct((n_rows, d), jnp.float32),
            jax.ShapeDtypeStruct((n_rows, _N_NODES), jnp.float32),
            jax.ShapeDtypeStruct((n_rows, 1), jnp.int32),
        ],
        scratch_shapes=[
            pltpu.VMEM((d, _N_NODES), jnp.float32),
            pltpu.VMEM((1, _N_NODES), jnp.float32),
        ],
    )(z_flat, tw_col, nodes_flat)

    som_z = som.reshape(b, t, d)
    bmu_indices = idx[:, 0].reshape(b, t)
    return som_z, q, bmu_indices


# speedup vs baseline: 7.8318x; 1.0563x over previous
import jax
import jax.numpy as jnp
from jax.experimental import pallas as pl
from jax.experimental.pallas import tpu as pltpu

_GRID = (32, 32)
_ALPHA = 1.0
_N_NODES = _GRID[0] * _GRID[1]
_BLK = 512

_DN_T = (((1,), (1,)), ((), ()))


def _som_block(z_ref, tw_ref, nodes_ref, som_ref, q_ref, idx_ref,
               nodes_t_ref, nn_ref):
    @pl.when(pl.program_id(0) == 0)
    def _prologue():
        nt = jnp.transpose(nodes_ref[...], (1, 0))
        nodes_t_ref[...] = nt
        nn_ref[...] = jnp.sum(nt * nt, axis=0, keepdims=True)

    z = z_ref[...]
    tw = tw_ref[...]
    nodes_t = nodes_t_ref[...]
    wz = z * tw

    mm = jnp.dot(wz, nodes_t, preferred_element_type=jnp.float32)
    rowsq = jnp.sum(wz * wz, axis=1, keepdims=True)
    sq = rowsq - 2.0 * mm + nn_ref[...]
    dists = jnp.sqrt(jnp.maximum(sq, 1e-12))

    q_raw = 1.0 / (1.0 + dists / _ALPHA)
    q_ref[...] = q_raw / jnp.sum(q_raw, axis=1, keepdims=True)

    idx = jnp.argmin(dists, axis=1).astype(jnp.int32)
    idx_col = idx[:, None]
    idx_ref[...] = idx[None, None, :]

    lane = jax.lax.broadcasted_iota(jnp.int32, dists.shape, 1)
    onehot = (lane == idx_col).astype(jnp.float32)
    gathered = jax.lax.dot_general(onehot, nodes_t, _DN_T,
                                   preferred_element_type=jnp.float32)
    som_ref[...] = 0.9 * z + 0.1 * gathered


@jax.jit
def kernel(z, nodes, time_weights):
    b, t, d = z.shape
    n_rows = b * t
    z_flat = z.reshape(n_rows, d)
    nodes_flat = nodes.reshape(-1, d)
    tw_col = time_weights[0, -t:, :]

    n_blocks = n_rows // _BLK
    tw_blocks = t // _BLK if t >= _BLK else 1

    som, q, idx = pl.pallas_call(
        _som_block,
        grid=(n_blocks,),
        in_specs=[
            pl.BlockSpec((_BLK, d), lambda i: (i, 0)),
            pl.BlockSpec((_BLK, 1), lambda i: (i % tw_blocks, 0)),
            pl.BlockSpec((_N_NODES, d), lambda i: (0, 0)),
        ],
        out_specs=[
            pl.BlockSpec((_BLK, d), lambda i: (i, 0)),
            pl.BlockSpec((_BLK, _N_NODES), lambda i: (i, 0)),
            pl.BlockSpec((1, 1, _BLK), lambda i: (i, 0, 0)),
        ],
        out_shape=[
            jax.ShapeDtypeStruct((n_rows, d), jnp.float32),
            jax.ShapeDtypeStruct((n_rows, _N_NODES), jnp.float32),
            jax.ShapeDtypeStruct((n_blocks, 1, _BLK), jnp.int32),
        ],
        scratch_shapes=[
            pltpu.VMEM((d, _N_NODES), jnp.float32),
            pltpu.VMEM((1, _N_NODES), jnp.float32),
        ],
    )(z_flat, tw_col, nodes_flat)

    som_z = som.reshape(b, t, d)
    bmu_indices = idx.reshape(b, t)
    return som_z, q, bmu_indices


# speedup vs baseline: 8.2747x; 1.0566x over previous
import functools

import jax
import jax.numpy as jnp
from jax.experimental import pallas as pl
from jax.experimental.pallas import tpu as pltpu

_GRID = (32, 32)
_ALPHA = 1.0
_N_NODES = _GRID[0] * _GRID[1]
_BLK = 512

_DN_T = (((1,), (1,)), ((), ()))


def _som_block(tw_base, z_ref, tw_ref, nodes_ref, som_ref, q_ref, idx_ref,
               nodes_t_ref, nn_ref, tw_col_ref):
    @pl.when(pl.program_id(0) == 0)
    def _prologue():
        nt = jnp.transpose(nodes_ref[...], (1, 0))
        nodes_t_ref[...] = nt
        nn_ref[...] = jnp.sum(nt * nt, axis=0, keepdims=True)
        tw_row = tw_ref[:, pl.ds(tw_base, _BLK)]
        tw_col_ref[...] = jnp.transpose(tw_row, (1, 0))

    z = z_ref[...]
    tw = tw_col_ref[...]
    nodes_t = nodes_t_ref[...]
    wz = z * tw

    mm = jnp.dot(wz, nodes_t, preferred_element_type=jnp.float32)
    rowsq = jnp.sum(wz * wz, axis=1, keepdims=True)
    sq = rowsq - 2.0 * mm + nn_ref[...]
    dists = jnp.sqrt(jnp.maximum(sq, 1e-12))

    q_raw = 1.0 / (1.0 + dists / _ALPHA)
    q_ref[...] = q_raw / jnp.sum(q_raw, axis=1, keepdims=True)

    idx = jnp.argmin(dists, axis=1).astype(jnp.int32)
    idx_col = idx[:, None]
    idx_ref[...] = idx[None, None, :]

    lane = jax.lax.broadcasted_iota(jnp.int32, dists.shape, 1)
    onehot = (lane == idx_col).astype(jnp.float32)
    gathered = jax.lax.dot_general(onehot, nodes_t, _DN_T,
                                   preferred_element_type=jnp.float32)
    som_ref[...] = 0.9 * z + 0.1 * gathered


@jax.jit
def kernel(z, nodes, time_weights):
    b, t, d = z.shape
    n_rows = b * t
    z_flat = z.reshape(n_rows, d)
    nodes_flat = nodes.reshape(-1, d)
    max_seq = time_weights.shape[1]
    tw_row = time_weights.reshape(1, max_seq)
    assert t == _BLK, "row blocks must align with the sequence length"

    n_blocks = n_rows // _BLK

    som, q, idx = pl.pallas_call(
        functools.partial(_som_block, max_seq - t),
        grid=(n_blocks,),
        in_specs=[
            pl.BlockSpec((_BLK, d), lambda i: (i, 0)),
            pl.BlockSpec((1, max_seq), lambda i: (0, 0)),
            pl.BlockSpec((_N_NODES, d), lambda i: (0, 0)),
        ],
        out_specs=[
            pl.BlockSpec((_BLK, d), lambda i: (i, 0)),
            pl.BlockSpec((_BLK, _N_NODES), lambda i: (i, 0)),
            pl.BlockSpec((1, 1, _BLK), lambda i: (i, 0, 0)),
        ],
        out_shape=[
            jax.ShapeDtypeStruct((n_rows, d), jnp.float32),
            jax.ShapeDtypeStruct((n_rows, _N_NODES), jnp.float32),
            jax.ShapeDtypeStruct((n_blocks, 1, _BLK), jnp.int32),
        ],
        scratch_shapes=[
            pltpu.VMEM((d, _N_NODES), jnp.float32),
            pltpu.VMEM((1, _N_NODES), jnp.float32),
            pltpu.VMEM((_BLK, 1), jnp.float32),
        ],
    )(z_flat, tw_row, nodes_flat)

    som_z = som.reshape(b, t, d)
    bmu_indices = idx.reshape(b, t)
    return som_z, q, bmu_indices
